# Initial kernel scaffold; baseline (speedup 1.0000x reference)
import jax
import jax.numpy as jnp
from jax.experimental import pallas as pl


def kernel(x, edge_index, batch, bnf_w, bnf_b, Wf, bf, W0, b0, bn0_w, bn0_b, W1, b1, bn1_w, bn1_b, W2, b2, bn2_w, bn2_b, G1, G1b, G2, G2b, F1, F1b, bnfc_w, bnfc_b, Wc, bc):
    raise NotImplementedError("write your pallas kernel here")


# speedup vs baseline: 10.3046x; 10.3046x over previous
import functools

import jax
import jax.numpy as jnp
from jax import lax
from jax.experimental import pallas as pl
from jax.experimental.pallas import tpu as pltpu
from jax.experimental.pallas import tpu_sc as plsc

N = 10000
E = 320000
D = 128
H = 128
C = 10
G = 64
EPS = 1e-5

NC = 2
NS = 16
NW = NC * NS
CH = 128
T = -(-E // (NW * CH))
EPAD = NW * T * CH
NTRASH = 112
NPAD = N + NTRASH
RPW = NPAD // NS

_MESH = plsc.VectorSubcoreMesh(core_axis_name="c", subcore_axis_name="s")


def _sc_scatter_body(y_hbm, src_hbm, dst_hbm, zeros_hbm, out_hbm,
                     src_v, dst_v, rows_v, acc_sh, sem):
    c = lax.axis_index("c")
    s = lax.axis_index("s")
    wid = s * NC + c
    pltpu.sync_copy(zeros_hbm, acc_sh.at[pl.ds(s * RPW, RPW)])
    plsc.subcore_barrier()

    def step(t, carry):
        pltpu.sync_copy(src_hbm.at[wid, t], src_v)
        pltpu.sync_copy(dst_hbm.at[wid, t], dst_v)
        pltpu.async_copy(y_hbm.at[src_v], rows_v, sem).wait()
        pltpu.sync_copy(rows_v, acc_sh.at[dst_v], add=True)
        return carry

    lax.fori_loop(0, T, step, 0)
    plsc.subcore_barrier()
    pltpu.sync_copy(acc_sh.at[pl.ds(s * RPW, RPW)],
                    out_hbm.at[c, pl.ds(s * RPW, RPW)])


_sc_scatter = pl.kernel(
    _sc_scatter_body,
    out_type=jax.ShapeDtypeStruct((NC, NPAD, D), jnp.float32),
    mesh=_MESH,
    scratch_types=[
        pltpu.VMEM((CH,), jnp.int32),
        pltpu.VMEM((CH,), jnp.int32),
        pltpu.VMEM((CH, D), jnp.float32),
        pltpu.VMEM_SHARED((NPAD, D), jnp.float32),
        pltpu.SemaphoreType.DMA,
    ],
)


def _bn_in(h, w, b):
    m = jnp.mean(h, axis=0)
    v = jnp.mean((h - m) ** 2, axis=0)
    return (h - m) / jnp.sqrt(v + EPS) * w + b


def _tc_a_body(x_ref, dega_ref, bnw_ref, bnb_ref, w_ref, y_ref, dis_ref):
    deg = dega_ref[0, :N, 0] + dega_ref[1, :N, 0] + 1.0
    dis = lax.rsqrt(deg)
    h = _bn_in(x_ref[...], bnw_ref[...], bnb_ref[...])
    xw = jnp.dot(h, w_ref[...], preferred_element_type=jnp.float32)
    y_ref[...] = xw * dis[:, None]
    dis_ref[...] = dis


def _tc_b_body(acc_ref, y_ref, dis_ref, bprev_ref, bnw_ref, bnb_ref, w_ref,
               yout_ref):
    dis = dis_ref[...]
    tot = acc_ref[0, :N, :] + acc_ref[1, :N, :] + y_ref[...]
    h = jnp.maximum(dis[:, None] * tot + bprev_ref[...], 0.0)
    hb = _bn_in(h, bnw_ref[...], bnb_ref[...])
    xw = jnp.dot(hb, w_ref[...], preferred_element_type=jnp.float32)
    yout_ref[...] = xw * dis[:, None]


def _tc_c_body(acc_ref, y_ref, dis_ref, bprev_ref, batch_ref, g1_ref, g1b_ref,
               g2_ref, g2b_ref, f1_ref, f1b_ref, bnw_ref, bnb_ref, wc_ref,
               bc_ref, out_ref):
    dis = dis_ref[...]
    tot = acc_ref[0, :N, :] + acc_ref[1, :N, :] + y_ref[...]
    h = jnp.maximum(dis[:, None] * tot + bprev_ref[...], 0.0)
    hg1 = jnp.maximum(
        jnp.dot(h, g1_ref[...], preferred_element_type=jnp.float32)
        + g1b_ref[...], 0.0)
    glin = jnp.dot(hg1, g2_ref[...], preferred_element_type=jnp.float32) \
        + g2b_ref[...]
    gate = jax.nn.sigmoid(glin)
    hg = h * gate
    onehot = (batch_ref[...][None, :]
              == lax.broadcasted_iota(jnp.int32, (G, N), 0)
              ).astype(jnp.float32)
    pooled = jnp.dot(onehot, hg, preferred_element_type=jnp.float32)
    pb = _bn_in(pooled, bnw_ref[...], bnb_ref[...])
    z = jnp.maximum(
        jnp.dot(pb, f1_ref[...], preferred_element_type=jnp.float32)
        + f1b_ref[...], 0.0)
    logits = jnp.dot(z, wc_ref[...], preferred_element_type=jnp.float32) \
        + bc_ref[...]
    lmax = jnp.max(logits, axis=-1, keepdims=True)
    e = logits - lmax
    out_ref[...] = e - jnp.log(jnp.sum(jnp.exp(e), axis=-1, keepdims=True))


def kernel(x, edge_index, batch, bnf_w, bnf_b, Wf, bf, W0, b0, bn0_w, bn0_b,
           W1, b1, bn1_w, bn1_b, W2, b2, bn2_w, bn2_b, G1, G1b, G2, G2b, F1,
           F1b, bnfc_w, bnfc_b, Wc, bc):
    src = edge_index[0].astype(jnp.int32)
    dst = edge_index[1].astype(jnp.int32)
    pad = EPAD - E
    pad_ar = jnp.arange(pad, dtype=jnp.int32)
    src_p = jnp.concatenate([src, pad_ar % 128]).reshape(NW, T, CH)
    dst_p = jnp.concatenate([dst, N + pad_ar % NTRASH]).reshape(NW, T, CH)

    zeros_d = jnp.zeros((RPW, D), jnp.float32)

    dega = _sc_scatter(jnp.ones((N, D), jnp.float32), src_p, dst_p, zeros_d)

    y0, dis = pl.pallas_call(
        _tc_a_body,
        out_shape=[jax.ShapeDtypeStruct((N, D), jnp.float32),
                   jax.ShapeDtypeStruct((N,), jnp.float32)],
    )(x, dega, bnf_w, bnf_b, Wf)

    tc_b = pl.pallas_call(
        _tc_b_body,
        out_shape=jax.ShapeDtypeStruct((N, D), jnp.float32),
    )

    y = y0
    for b_prev, bw, bb, W in ((bf, bn0_w, bn0_b, W0),
                              (b0, bn1_w, bn1_b, W1),
                              (b1, bn2_w, bn2_b, W2)):
        acc = _sc_scatter(y, src_p, dst_p, zeros_d)
        y = tc_b(acc, y, dis, b_prev, bw, bb, W)

    acc = _sc_scatter(y, src_p, dst_p, zeros_d)
    out = pl.pallas_call(
        _tc_c_body,
        out_shape=jax.ShapeDtypeStruct((G, C), jnp.float32),
    )(acc, y, dis, b2, batch, G1, G1b, G2, G2b, F1, F1b, bnfc_w, bnfc_b,
      Wc, bc)
    return out


# speedup vs baseline: 19.2617x; 1.8692x over previous
import jax
import jax.numpy as jnp
from jax import lax
from jax.experimental import pallas as pl
from jax.experimental.pallas import tpu as pltpu
from jax.experimental.pallas import tpu_sc as plsc

N = 10000
E = 320000
D = 128
H = 128
C = 10
G = 64
EPS = 1e-5

NC = 2
NS = 16
NW = NC * NS
CH = 112
NB = 3
T = -(-E // (NW * CH))
EPAD = NW * T * CH
NTRASH = 112
NPAD = N + NTRASH
RPW = NPAD // NS

_MESH = plsc.VectorSubcoreMesh(core_axis_name="c", subcore_axis_name="s")


def _sc_scatter_body(y_hbm, idx_hbm, zeros_hbm, out_hbm,
                     i0, i1, i2, r0, r1, r2, acc_sh, gsem, isem):
    c = lax.axis_index("c")
    s = lax.axis_index("s")
    wid = s * NC + c
    idx = (i0, i1, i2)
    rows = (r0, r1, r2)
    pltpu.sync_copy(zeros_hbm, acc_sh.at[pl.ds(s * RPW, RPW)])
    plsc.subcore_barrier()
    for b in range(NB):
        pltpu.async_copy(idx_hbm.at[wid, b], idx[b], isem)
    for b in range(2):
        pltpu.make_async_copy(idx_hbm.at[wid, b], idx[b], isem).wait()
        pltpu.async_copy(y_hbm.at[idx[b].at[0]], rows[b], gsem)

    def stage(t, b, b2):
        @pl.when(t + 2 < T)
        def _():
            pltpu.make_async_copy(idx_hbm.at[wid, 0], idx[b2], isem).wait()
            pltpu.async_copy(y_hbm.at[idx[b2].at[0]], rows[b2], gsem)
        pltpu.make_async_copy(y_hbm.at[idx[b].at[0]], rows[b], gsem).wait()
        pltpu.sync_copy(rows[b], acc_sh.at[idx[b].at[1]], add=True)
        pltpu.async_copy(idx_hbm.at[wid, t + 3], idx[b], isem)

    def outer(i, carry):
        for b in range(NB):
            t = i * NB + b
            stage(t, b, (b + 2) % NB)
        return carry

    lax.fori_loop(0, T // NB, outer, 0)
    for b in range(NB):
        pltpu.make_async_copy(idx_hbm.at[wid, 0], idx[b], isem).wait()
    plsc.subcore_barrier()
    pltpu.sync_copy(acc_sh.at[pl.ds(s * RPW, RPW)],
                    out_hbm.at[c, pl.ds(s * RPW, RPW)])


_sc_scatter = pl.kernel(
    _sc_scatter_body,
    out_type=jax.ShapeDtypeStruct((NC, NPAD, D), jnp.float32),
    mesh=_MESH,
    scratch_types=[
        pltpu.VMEM((2, CH), jnp.int32),
        pltpu.VMEM((2, CH), jnp.int32),
        pltpu.VMEM((2, CH), jnp.int32),
        pltpu.VMEM((CH, D), jnp.float32),
        pltpu.VMEM((CH, D), jnp.float32),
        pltpu.VMEM((CH, D), jnp.float32),
        pltpu.VMEM_SHARED((NPAD, D), jnp.float32),
        pltpu.SemaphoreType.DMA,
        pltpu.SemaphoreType.DMA,
    ],
)


def _sc_deg_body(idx_hbm, ones_hbm, zeros_hbm, out_hbm, i0, r0, acc_sh, isem):
    c = lax.axis_index("c")
    s = lax.axis_index("s")
    wid = s * NC + c
    pltpu.sync_copy(zeros_hbm, acc_sh.at[pl.ds(s * RPW, RPW)])
    pltpu.sync_copy(ones_hbm, r0)
    plsc.subcore_barrier()
    pltpu.async_copy(idx_hbm.at[wid, 0], i0, isem)

    def step(t, carry):
        pltpu.make_async_copy(idx_hbm.at[wid, 0], i0, isem).wait()
        pltpu.sync_copy(r0, acc_sh.at[i0.at[1]], add=True)

        @pl.when(t + 1 < T)
        def _():
            pltpu.async_copy(idx_hbm.at[wid, t + 1], i0, isem)
        return carry

    lax.fori_loop(0, T, step, 0)
    plsc.subcore_barrier()
    pltpu.sync_copy(acc_sh.at[pl.ds(s * RPW, RPW)],
                    out_hbm.at[c, pl.ds(s * RPW, RPW)])


_sc_deg = pl.kernel(
    _sc_deg_body,
    out_type=jax.ShapeDtypeStruct((NC, NPAD, D), jnp.float32),
    mesh=_MESH,
    scratch_types=[
        pltpu.VMEM((2, CH), jnp.int32),
        pltpu.VMEM((CH, D), jnp.float32),
        pltpu.VMEM_SHARED((NPAD, D), jnp.float32),
        pltpu.SemaphoreType.DMA,
    ],
)


def _bn_in(h, w, b):
    m = jnp.mean(h, axis=0)
    v = jnp.mean((h - m) ** 2, axis=0)
    return (h - m) / jnp.sqrt(v + EPS) * w + b


def _tc_a_body(x_ref, dega_ref, bnw_ref, bnb_ref, w_ref, y_ref, dis_ref):
    deg = dega_ref[0, :N, 0] + dega_ref[1, :N, 0] + 1.0
    dis = lax.rsqrt(deg)
    h = _bn_in(x_ref[...], bnw_ref[...], bnb_ref[...])
    xw = jnp.dot(h, w_ref[...], preferred_element_type=jnp.float32)
    y_ref[...] = xw * dis[:, None]
    dis_ref[...] = dis


def _tc_b_body(acc_ref, y_ref, dis_ref, bprev_ref, bnw_ref, bnb_ref, w_ref,
               yout_ref):
    dis = dis_ref[...]
    tot = acc_ref[0, :N, :] + acc_ref[1, :N, :] + y_ref[...]
    h = jnp.maximum(dis[:, None] * tot + bprev_ref[...], 0.0)
    hb = _bn_in(h, bnw_ref[...], bnb_ref[...])
    xw = jnp.dot(hb, w_ref[...], preferred_element_type=jnp.float32)
    yout_ref[...] = xw * dis[:, None]


def _tc_c_body(acc_ref, y_ref, dis_ref, bprev_ref, batch_ref, g1_ref, g1b_ref,
               g2_ref, g2b_ref, f1_ref, f1b_ref, bnw_ref, bnb_ref, wc_ref,
               bc_ref, out_ref):
    dis = dis_ref[...]
    tot = acc_ref[0, :N, :] + acc_ref[1, :N, :] + y_ref[...]
    h = jnp.maximum(dis[:, None] * tot + bprev_ref[...], 0.0)
    hg1 = jnp.maximum(
        jnp.dot(h, g1_ref[...], preferred_element_type=jnp.float32)
        + g1b_ref[...], 0.0)
    glin = jnp.dot(hg1, g2_ref[...], preferred_element_type=jnp.float32) \
        + g2b_ref[...]
    gate = jax.nn.sigmoid(glin)
    hg = h * gate
    onehot = (batch_ref[...][None, :]
              == lax.broadcasted_iota(jnp.int32, (G, N), 0)
              ).astype(jnp.float32)
    pooled = jnp.dot(onehot, hg, preferred_element_type=jnp.float32)
    pb = _bn_in(pooled, bnw_ref[...], bnb_ref[...])
    z = jnp.maximum(
        jnp.dot(pb, f1_ref[...], preferred_element_type=jnp.float32)
        + f1b_ref[...], 0.0)
    logits = jnp.dot(z, wc_ref[...], preferred_element_type=jnp.float32) \
        + bc_ref[...]
    lmax = jnp.max(logits, axis=-1, keepdims=True)
    e = logits - lmax
    out_ref[...] = e - jnp.log(jnp.sum(jnp.exp(e), axis=-1, keepdims=True))


def kernel(x, edge_index, batch, bnf_w, bnf_b, Wf, bf, W0, b0, bn0_w, bn0_b,
           W1, b1, bn1_w, bn1_b, W2, b2, bn2_w, bn2_b, G1, G1b, G2, G2b, F1,
           F1b, bnfc_w, bnfc_b, Wc, bc):
    src = edge_index[0].astype(jnp.int32)
    dst = edge_index[1].astype(jnp.int32)
    pad = EPAD - E
    pad_ar = jnp.arange(pad, dtype=jnp.int32)
    src_p = jnp.concatenate([src, pad_ar % 128]).reshape(NW, T, CH)
    dst_p = jnp.concatenate([dst, N + pad_ar % NTRASH]).reshape(NW, T, CH)
    idx_p = jnp.stack([src_p, dst_p], axis=2)
    idx_p = jnp.concatenate(
        [idx_p, jnp.zeros((NW, NB, 2, CH), jnp.int32)], axis=1)

    zeros_d = jnp.zeros((RPW, D), jnp.float32)
    ones_d = jnp.ones((CH, D), jnp.float32)

    dega = _sc_deg(idx_p, ones_d, zeros_d)

    y0, dis = pl.pallas_call(
        _tc_a_body,
        out_shape=[jax.ShapeDtypeStruct((N, D), jnp.float32),
                   jax.ShapeDtypeStruct((N,), jnp.float32)],
    )(x, dega, bnf_w, bnf_b, Wf)

    tc_b = pl.pallas_call(
        _tc_b_body,
        out_shape=jax.ShapeDtypeStruct((N, D), jnp.float32),
    )

    y = y0
    for b_prev, bw, bb, W in ((bf, bn0_w, bn0_b, W0),
                              (b0, bn1_w, bn1_b, W1),
                              (b1, bn2_w, bn2_b, W2)):
        acc = _sc_scatter(y, idx_p, zeros_d)
        y = tc_b(acc, y, dis, b_prev, bw, bb, W)

    acc = _sc_scatter(y, idx_p, zeros_d)
    out = pl.pallas_call(
        _tc_c_body,
        out_shape=jax.ShapeDtypeStruct((G, C), jnp.float32),
    )(acc, y, dis, b2, batch, G1, G1b, G2, G2b, F1, F1b, bnfc_w, bnfc_b,
      Wc, bc)
    return out


# speedup vs baseline: 23.1593x; 1.2023x over previous
import jax
import jax.numpy as jnp
from jax import lax
from jax.experimental import pallas as pl
from jax.experimental.pallas import tpu as pltpu
from jax.experimental.pallas import tpu_sc as plsc

N = 10000
E = 320000
D = 128
H = 128
C = 10
G = 64
EPS = 1e-5

NC = 2
NS = 16
NW = NC * NS
CH = 120
NB = 3
NI = 6
T = -(-E // (NW * CH))
EPAD = NW * T * CH
NTRASH = 112
NPAD = N + NTRASH
RPW = NPAD // NS

_MESH = plsc.VectorSubcoreMesh(core_axis_name="c", subcore_axis_name="s")


def _sc_scatter_body(y_hbm, idx_hbm, zeros_hbm, out_hbm,
                     i0, i1, i2, i3, i4, i5, r0, r1, r2, acc_sh,
                     gsem, isem, ssem):
    c = lax.axis_index("c")
    s = lax.axis_index("s")
    wid = s * NC + c
    idx = (i0, i1, i2, i3, i4, i5)
    rows = (r0, r1, r2)
    pltpu.sync_copy(zeros_hbm, acc_sh.at[pl.ds(s * RPW, RPW)])
    plsc.subcore_barrier()
    for b in range(5):
        pltpu.async_copy(idx_hbm.at[wid, b], idx[b], isem)
    for b in range(2):
        pltpu.make_async_copy(idx_hbm.at[wid, b], idx[b], isem).wait()
        pltpu.async_copy(y_hbm.at[idx[b].at[0]], rows[b], gsem)

    def stage(t, j):
        br, br2 = j % NB, (j + 2) % NB
        bi, bi2, bi5 = j, (j + 2) % NI, (j + 5) % NI
        cond = not isinstance(t, int)
        if cond or t + 2 < T:
            def g():
                pltpu.make_async_copy(idx_hbm.at[wid, 0], idx[bi2],
                                      isem).wait()
                if cond or t >= 1:
                    pltpu.make_async_copy(
                        rows[br2], acc_sh.at[idx[bi2].at[1]], ssem).wait()
                pltpu.async_copy(y_hbm.at[idx[bi2].at[0]], rows[br2], gsem)
            g()
        elif t >= 1:
            pltpu.make_async_copy(rows[br2], acc_sh.at[idx[bi2].at[1]],
                                  ssem).wait()
        pltpu.make_async_copy(y_hbm.at[idx[bi].at[0]], rows[br], gsem).wait()
        pltpu.async_copy(rows[br], acc_sh.at[idx[bi].at[1]], ssem, add=True)
        if cond or t + 5 < T:
            pltpu.async_copy(idx_hbm.at[wid, t + 5], idx[bi5], isem)

    for t in range(NI):
        stage(t, t)

    def middle(i, carry):
        for j in range(NI):
            stage(i * NI + j, j)
        return carry

    lax.fori_loop(1, T // NI - 1, middle, 0)
    for t in range(T - NI, T):
        stage(t, t % NI)
    pltpu.make_async_copy(rows[(T - 1) % NB],
                          acc_sh.at[idx[(T - 1) % NI].at[1]], ssem).wait()
    plsc.subcore_barrier()
    pltpu.sync_copy(acc_sh.at[pl.ds(s * RPW, RPW)],
                    out_hbm.at[c, pl.ds(s * RPW, RPW)])


_sc_scatter = pl.kernel(
    _sc_scatter_body,
    out_type=jax.ShapeDtypeStruct((NC, NPAD, D), jnp.float32),
    mesh=_MESH,
    scratch_types=(
        [pltpu.VMEM((2, CH), jnp.int32)] * NI
        + [pltpu.VMEM((CH, D), jnp.float32)] * NB
        + [pltpu.VMEM_SHARED((NPAD, D), jnp.float32),
           pltpu.SemaphoreType.DMA,
           pltpu.SemaphoreType.DMA,
           pltpu.SemaphoreType.DMA]
    ),
)


def _sc_deg_body(idx_hbm, ones_hbm, zeros_hbm, out_hbm, i0, r0, acc_sh, isem):
    c = lax.axis_index("c")
    s = lax.axis_index("s")
    wid = s * NC + c
    pltpu.sync_copy(zeros_hbm, acc_sh.at[pl.ds(s * RPW, RPW)])
    pltpu.sync_copy(ones_hbm, r0)
    plsc.subcore_barrier()
    pltpu.async_copy(idx_hbm.at[wid, 0], i0, isem)

    def step(t, carry):
        pltpu.make_async_copy(idx_hbm.at[wid, 0], i0, isem).wait()
        pltpu.sync_copy(r0, acc_sh.at[i0.at[1]], add=True)

        @pl.when(t + 1 < T)
        def _():
            pltpu.async_copy(idx_hbm.at[wid, t + 1], i0, isem)
        return carry

    lax.fori_loop(0, T, step, 0)
    plsc.subcore_barrier()
    pltpu.sync_copy(acc_sh.at[pl.ds(s * RPW, RPW)],
                    out_hbm.at[c, pl.ds(s * RPW, RPW)])


_sc_deg = pl.kernel(
    _sc_deg_body,
    out_type=jax.ShapeDtypeStruct((NC, NPAD, D), jnp.float32),
    mesh=_MESH,
    scratch_types=[
        pltpu.VMEM((2, CH), jnp.int32),
        pltpu.VMEM((CH, D), jnp.float32),
        pltpu.VMEM_SHARED((NPAD, D), jnp.float32),
        pltpu.SemaphoreType.DMA,
    ],
)


def _bn_in(h, w, b):
    m = jnp.mean(h, axis=0)
    v = jnp.mean((h - m) ** 2, axis=0)
    return (h - m) / jnp.sqrt(v + EPS) * w + b


def _tc_a_body(x_ref, dega_ref, bnw_ref, bnb_ref, w_ref, y_ref, dis_ref):
    deg = dega_ref[0, :N, 0] + dega_ref[1, :N, 0] + 1.0
    dis = lax.rsqrt(deg)
    h = _bn_in(x_ref[...], bnw_ref[...], bnb_ref[...])
    xw = jnp.dot(h, w_ref[...], preferred_element_type=jnp.float32)
    y_ref[...] = xw * dis[:, None]
    dis_ref[...] = dis


def _tc_b_body(acc_ref, y_ref, dis_ref, bprev_ref, bnw_ref, bnb_ref, w_ref,
               yout_ref):
    dis = dis_ref[...]
    tot = acc_ref[0, :N, :] + acc_ref[1, :N, :] + y_ref[...]
    h = jnp.maximum(dis[:, None] * tot + bprev_ref[...], 0.0)
    hb = _bn_in(h, bnw_ref[...], bnb_ref[...])
    xw = jnp.dot(hb, w_ref[...], preferred_element_type=jnp.float32)
    yout_ref[...] = xw * dis[:, None]


def _tc_c_body(acc_ref, y_ref, dis_ref, bprev_ref, batch_ref, g1_ref, g1b_ref,
               g2_ref, g2b_ref, f1_ref, f1b_ref, bnw_ref, bnb_ref, wc_ref,
               bc_ref, out_ref):
    dis = dis_ref[...]
    tot = acc_ref[0, :N, :] + acc_ref[1, :N, :] + y_ref[...]
    h = jnp.maximum(dis[:, None] * tot + bprev_ref[...], 0.0)
    hg1 = jnp.maximum(
        jnp.dot(h, g1_ref[...], preferred_element_type=jnp.float32)
        + g1b_ref[...], 0.0)
    glin = jnp.dot(hg1, g2_ref[...], preferred_element_type=jnp.float32) \
        + g2b_ref[...]
    gate = jax.nn.sigmoid(glin)
    hg = h * gate
    onehot = (batch_ref[...][None, :]
              == lax.broadcasted_iota(jnp.int32, (G, N), 0)
              ).astype(jnp.float32)
    pooled = jnp.dot(onehot, hg, preferred_element_type=jnp.float32)
    pb = _bn_in(pooled, bnw_ref[...], bnb_ref[...])
    z = jnp.maximum(
        jnp.dot(pb, f1_ref[...], preferred_element_type=jnp.float32)
        + f1b_ref[...], 0.0)
    logits = jnp.dot(z, wc_ref[...], preferred_element_type=jnp.float32) \
        + bc_ref[...]
    lmax = jnp.max(logits, axis=-1, keepdims=True)
    e = logits - lmax
    out_ref[...] = e - jnp.log(jnp.sum(jnp.exp(e), axis=-1, keepdims=True))


def kernel(x, edge_index, batch, bnf_w, bnf_b, Wf, bf, W0, b0, bn0_w, bn0_b,
           W1, b1, bn1_w, bn1_b, W2, b2, bn2_w, bn2_b, G1, G1b, G2, G2b, F1,
           F1b, bnfc_w, bnfc_b, Wc, bc):
    src = edge_index[0].astype(jnp.int32)
    dst = edge_index[1].astype(jnp.int32)
    pad = EPAD - E
    pad_ar = jnp.arange(pad, dtype=jnp.int32)
    src_p = jnp.concatenate([src, pad_ar % 128]).reshape(NW, T, CH)
    dst_p = jnp.concatenate([dst, N + pad_ar % NTRASH]).reshape(NW, T, CH)
    idx_p = jnp.stack([src_p, dst_p], axis=2)

    zeros_d = jnp.zeros((RPW, D), jnp.float32)
    ones_d = jnp.ones((CH, D), jnp.float32)

    dega = _sc_deg(idx_p, ones_d, zeros_d)

    y0, dis = pl.pallas_call(
        _tc_a_body,
        out_shape=[jax.ShapeDtypeStruct((N, D), jnp.float32),
                   jax.ShapeDtypeStruct((N,), jnp.float32)],
    )(x, dega, bnf_w, bnf_b, Wf)

    tc_b = pl.pallas_call(
        _tc_b_body,
        out_shape=jax.ShapeDtypeStruct((N, D), jnp.float32),
    )

    y = y0
    for b_prev, bw, bb, W in ((bf, bn0_w, bn0_b, W0),
                              (b0, bn1_w, bn1_b, W1),
                              (b1, bn2_w, bn2_b, W2)):
        acc = _sc_scatter(y, idx_p, zeros_d)
        y = tc_b(acc, y, dis, b_prev, bw, bb, W)

    acc = _sc_scatter(y, idx_p, zeros_d)
    out = pl.pallas_call(
        _tc_c_body,
        out_shape=jax.ShapeDtypeStruct((G, C), jnp.float32),
    )(acc, y, dis, b2, batch, G1, G1b, G2, G2b, F1, F1b, bnfc_w, bnfc_b,
      Wc, bc)
    return out


# speedup vs baseline: 26.9484x; 1.1636x over previous
import jax
import jax.numpy as jnp
from jax import lax
from jax.experimental import pallas as pl
from jax.experimental.pallas import tpu as pltpu
from jax.experimental.pallas import tpu_sc as plsc

N = 10000
E = 320000
D = 128
H = 128
C = 10
G = 64
EPS = 1e-5

NC = 2
NS = 16
NW = NC * NS
CH = 120
NB = 3
NI = 6
T = -(-E // (NW * CH))
EPAD = NW * T * CH
NTRASH = 240
NPAD = N + NTRASH
RPW = NPAD // NS

_MESH = plsc.VectorSubcoreMesh(core_axis_name="c", subcore_axis_name="s")


def _sc_scatter_body(y_hbm, idx_hbm, zeros_hbm, out_hbm,
                     i0, i1, i2, i3, i4, i5, r0, r1, r2, acc_sh,
                     gsem, isem, ssem):
    c = lax.axis_index("c")
    s = lax.axis_index("s")
    wid = s * NC + c
    idx = (i0, i1, i2, i3, i4, i5)
    rows = (r0, r1, r2)
    pltpu.sync_copy(zeros_hbm, acc_sh.at[pl.ds(s * RPW, RPW)])
    plsc.subcore_barrier()
    for b in range(5):
        pltpu.async_copy(idx_hbm.at[wid, b], idx[b], isem)
    for b in range(2):
        pltpu.make_async_copy(idx_hbm.at[wid, b], idx[b], isem).wait()
        pltpu.async_copy(y_hbm.at[idx[b].at[0]], rows[b], gsem)

    def stage(t, j):
        br, br2 = j % NB, (j + 2) % NB
        bi, bi2, bi5 = j, (j + 2) % NI, (j + 5) % NI
        cond = not isinstance(t, int)
        if cond or t + 2 < T:
            def g():
                pltpu.make_async_copy(idx_hbm.at[wid, 0], idx[bi2],
                                      isem).wait()
                if cond or t >= 1:
                    pltpu.make_async_copy(
                        rows[br2], acc_sh.at[idx[bi2].at[1]], ssem).wait()
                pltpu.async_copy(y_hbm.at[idx[bi2].at[0]], rows[br2], gsem)
            g()
        elif t >= 1:
            pltpu.make_async_copy(rows[br2], acc_sh.at[idx[bi2].at[1]],
                                  ssem).wait()
        pltpu.make_async_copy(y_hbm.at[idx[bi].at[0]], rows[br], gsem).wait()
        pltpu.async_copy(rows[br], acc_sh.at[idx[bi].at[1]], ssem, add=True)
        if cond or t + 5 < T:
            pltpu.async_copy(idx_hbm.at[wid, t + 5], idx[bi5], isem)

    for t in range(NI):
        stage(t, t)

    def middle(i, carry):
        for j in range(NI):
            stage(i * NI + j, j)
        return carry

    lax.fori_loop(1, T // NI - 1, middle, 0)
    for t in range(T - NI, T):
        stage(t, t % NI)
    pltpu.make_async_copy(rows[(T - 1) % NB],
                          acc_sh.at[idx[(T - 1) % NI].at[1]], ssem).wait()
    plsc.subcore_barrier()
    pltpu.sync_copy(acc_sh.at[pl.ds(s * RPW, RPW)],
                    out_hbm.at[c, pl.ds(s * RPW, RPW)])


_sc_scatter = pl.kernel(
    _sc_scatter_body,
    out_type=jax.ShapeDtypeStruct((NC, NPAD, D), jnp.float32),
    mesh=_MESH,
    scratch_types=(
        [pltpu.VMEM((2, CH), jnp.int32)] * NI
        + [pltpu.VMEM((CH, D), jnp.float32)] * NB
        + [pltpu.VMEM_SHARED((NPAD, D), jnp.float32),
           pltpu.SemaphoreType.DMA,
           pltpu.SemaphoreType.DMA,
           pltpu.SemaphoreType.DMA]
    ),
)


def _sc_deg_body(idx_hbm, ones_hbm, zeros_hbm, out_hbm, i0, i1, ones_v,
                 acc_sh, isem):
    c = lax.axis_index("c")
    s = lax.axis_index("s")
    wid = s * NC + c
    idx = (i0, i1)
    pltpu.sync_copy(zeros_hbm, acc_sh.at[pl.ds(s * RPW, RPW)])
    pltpu.sync_copy(ones_hbm, ones_v)
    plsc.subcore_barrier()
    pltpu.async_copy(idx_hbm.at[wid, 0], i0, isem)
    pltpu.async_copy(idx_hbm.at[wid, 1], i1, isem)

    def stage(t, j):
        pltpu.make_async_copy(idx_hbm.at[wid, 0], idx[j], isem).wait()
        pltpu.sync_copy(ones_v, acc_sh.at[idx[j].at[1]], add=True)
        cond = not isinstance(t, int)
        if cond or t + 2 < T:
            pltpu.async_copy(idx_hbm.at[wid, t + 2], idx[j], isem)

    for t in range(2):
        stage(t, t)

    def middle(i, carry):
        for j in range(2):
            stage(i * 2 + j, j)
        return carry

    lax.fori_loop(1, T // 2 - 1, middle, 0)
    for t in range(T - 2, T):
        stage(t, t % 2)
    plsc.subcore_barrier()
    pltpu.sync_copy(acc_sh.at[pl.ds(s * RPW, RPW)],
                    out_hbm.at[pl.ds(c * NPAD + s * RPW, RPW)])


_sc_deg = pl.kernel(
    _sc_deg_body,
    out_type=jax.ShapeDtypeStruct((NC * NPAD,), jnp.float32),
    mesh=_MESH,
    scratch_types=[
        pltpu.VMEM((2, CH), jnp.int32),
        pltpu.VMEM((2, CH), jnp.int32),
        pltpu.VMEM((CH,), jnp.float32),
        pltpu.VMEM_SHARED((NPAD,), jnp.float32),
        pltpu.SemaphoreType.DMA,
    ],
)


def _bn_in(h, w, b):
    m = jnp.mean(h, axis=0)
    v = jnp.mean((h - m) ** 2, axis=0)
    return (h - m) / jnp.sqrt(v + EPS) * w + b


def _tc_a_body(x_ref, dega_ref, bnw_ref, bnb_ref, w_ref, y_ref, dis_ref):
    deg = dega_ref[0, :N] + dega_ref[1, :N] + 1.0
    dis = lax.rsqrt(deg)
    h = _bn_in(x_ref[...], bnw_ref[...], bnb_ref[...])
    xw = jnp.dot(h, w_ref[...], preferred_element_type=jnp.float32)
    y_ref[...] = xw * dis[:, None]
    dis_ref[...] = dis


def _tc_b_body(acc_ref, y_ref, dis_ref, bprev_ref, bnw_ref, bnb_ref, w_ref,
               yout_ref):
    dis = dis_ref[...]
    tot = acc_ref[0, :N, :] + acc_ref[1, :N, :] + y_ref[...]
    h = jnp.maximum(dis[:, None] * tot + bprev_ref[...], 0.0)
    hb = _bn_in(h, bnw_ref[...], bnb_ref[...])
    xw = jnp.dot(hb, w_ref[...], preferred_element_type=jnp.float32)
    yout_ref[...] = xw * dis[:, None]


def _tc_c_body(acc_ref, y_ref, dis_ref, bprev_ref, batch_ref, g1_ref, g1b_ref,
               g2_ref, g2b_ref, f1_ref, f1b_ref, bnw_ref, bnb_ref, wc_ref,
               bc_ref, out_ref):
    dis = dis_ref[...]
    tot = acc_ref[0, :N, :] + acc_ref[1, :N, :] + y_ref[...]
    h = jnp.maximum(dis[:, None] * tot + bprev_ref[...], 0.0)
    hg1 = jnp.maximum(
        jnp.dot(h, g1_ref[...], preferred_element_type=jnp.float32)
        + g1b_ref[...], 0.0)
    glin = jnp.dot(hg1, g2_ref[...], preferred_element_type=jnp.float32) \
        + g2b_ref[...]
    gate = jax.nn.sigmoid(glin)
    hg = h * gate
    onehot = (batch_ref[...][None, :]
              == lax.broadcasted_iota(jnp.int32, (G, N), 0)
              ).astype(jnp.float32)
    pooled = jnp.dot(onehot, hg, preferred_element_type=jnp.float32)
    pb = _bn_in(pooled, bnw_ref[...], bnb_ref[...])
    z = jnp.maximum(
        jnp.dot(pb, f1_ref[...], preferred_element_type=jnp.float32)
        + f1b_ref[...], 0.0)
    logits = jnp.dot(z, wc_ref[...], preferred_element_type=jnp.float32) \
        + bc_ref[...]
    lmax = jnp.max(logits, axis=-1, keepdims=True)
    e = logits - lmax
    out_ref[...] = e - jnp.log(jnp.sum(jnp.exp(e), axis=-1, keepdims=True))


def kernel(x, edge_index, batch, bnf_w, bnf_b, Wf, bf, W0, b0, bn0_w, bn0_b,
           W1, b1, bn1_w, bn1_b, W2, b2, bn2_w, bn2_b, G1, G1b, G2, G2b, F1,
           F1b, bnfc_w, bnfc_b, Wc, bc):
    src = edge_index[0].astype(jnp.int32)
    dst = edge_index[1].astype(jnp.int32)
    pad = EPAD - E
    pad_ar = jnp.arange(pad, dtype=jnp.int32)
    src_p = jnp.concatenate([src, pad_ar % 128]).reshape(NW, T, CH)
    dst_p = jnp.concatenate([dst, N + pad_ar % NTRASH]).reshape(NW, T, CH)
    idx_p = jnp.stack([src_p, dst_p], axis=2)

    zeros_d = jnp.zeros((RPW, D), jnp.float32)

    dega = _sc_deg(idx_p, jnp.ones((CH,), jnp.float32),
                   jnp.zeros((RPW,), jnp.float32)).reshape(NC, NPAD)

    y0, dis = pl.pallas_call(
        _tc_a_body,
        out_shape=[jax.ShapeDtypeStruct((N, D), jnp.float32),
                   jax.ShapeDtypeStruct((N,), jnp.float32)],
    )(x, dega, bnf_w, bnf_b, Wf)

    tc_b = pl.pallas_call(
        _tc_b_body,
        out_shape=jax.ShapeDtypeStruct((N, D), jnp.float32),
    )

    y = y0
    for b_prev, bw, bb, W in ((bf, bn0_w, bn0_b, W0),
                              (b0, bn1_w, bn1_b, W1),
                              (b1, bn2_w, bn2_b, W2)):
        acc = _sc_scatter(y, idx_p, zeros_d)
        y = tc_b(acc, y, dis, b_prev, bw, bb, W)

    acc = _sc_scatter(y, idx_p, zeros_d)
    out = pl.pallas_call(
        _tc_c_body,
        out_shape=jax.ShapeDtypeStruct((G, C), jnp.float32),
    )(acc, y, dis, b2, batch, G1, G1b, G2, G2b, F1, F1b, bnfc_w, bnfc_b,
      Wc, bc)
    return out


# speedup vs baseline: 27.2202x; 1.0101x over previous
import jax
import jax.numpy as jnp
from jax import lax
from jax.experimental import pallas as pl
from jax.experimental.pallas import tpu as pltpu
from jax.experimental.pallas import tpu_sc as plsc

N = 10000
E = 320000
D = 128
H = 128
C = 10
G = 64
EPS = 1e-5

NC = 2
NS = 16
NW = NC * NS
CH = 120
NB = 3
NI = 6
T = -(-E // (NW * CH))
EPAD = NW * T * CH
NTRASH = 240
NPAD = N + NTRASH
RPW = NPAD // NS

_MESH = plsc.VectorSubcoreMesh(core_axis_name="c", subcore_axis_name="s")


def _sc_scatter_body(y_hbm, idx_hbm, zeros_hbm, out_hbm,
                     i0, i1, i2, i3, i4, i5, r0, r1, r2, acc_sh,
                     gsem, isem, ssem):
    c = lax.axis_index("c")
    s = lax.axis_index("s")
    wid = s * NC + c
    idx = (i0, i1, i2, i3, i4, i5)
    rows = (r0, r1, r2)
    pltpu.sync_copy(zeros_hbm.at[s], acc_sh.at[pl.ds(s * RPW, RPW)])
    plsc.subcore_barrier()
    for b in range(5):
        pltpu.async_copy(idx_hbm.at[wid, b], idx[b], isem)
    for b in range(2):
        pltpu.make_async_copy(idx_hbm.at[wid, b], idx[b], isem).wait()
        pltpu.async_copy(y_hbm.at[idx[b].at[0]], rows[b], gsem)

    def stage(t, j):
        br, br2 = j % NB, (j + 2) % NB
        bi, bi2, bi5 = j, (j + 2) % NI, (j + 5) % NI
        cond = not isinstance(t, int)
        if cond or t + 2 < T:
            def g():
                pltpu.make_async_copy(idx_hbm.at[wid, 0], idx[bi2],
                                      isem).wait()
                if cond or t >= 1:
                    pltpu.make_async_copy(
                        rows[br2], acc_sh.at[idx[bi2].at[1]], ssem).wait()
                pltpu.async_copy(y_hbm.at[idx[bi2].at[0]], rows[br2], gsem)
            g()
        elif t >= 1:
            pltpu.make_async_copy(rows[br2], acc_sh.at[idx[bi2].at[1]],
                                  ssem).wait()
        pltpu.make_async_copy(y_hbm.at[idx[bi].at[0]], rows[br], gsem).wait()
        pltpu.async_copy(rows[br], acc_sh.at[idx[bi].at[1]], ssem, add=True)
        if cond or t + 5 < T:
            pltpu.async_copy(idx_hbm.at[wid, t + 5], idx[bi5], isem)

    for t in range(NI):
        stage(t, t)

    def middle(i, carry):
        for j in range(NI):
            stage(i * NI + j, j)
        return carry

    lax.fori_loop(1, T // NI - 1, middle, 0)
    for t in range(T - NI, T):
        stage(t, t % NI)
    pltpu.make_async_copy(rows[(T - 1) % NB],
                          acc_sh.at[idx[(T - 1) % NI].at[1]], ssem).wait()
    plsc.subcore_barrier()
    pltpu.sync_copy(acc_sh.at[pl.ds(s * RPW, RPW)],
                    out_hbm.at[c, pl.ds(s * RPW, RPW)])


_sc_scatter = pl.kernel(
    _sc_scatter_body,
    out_type=jax.ShapeDtypeStruct((NC, NPAD, D), jnp.float32),
    mesh=_MESH,
    scratch_types=(
        [pltpu.VMEM((2, CH), jnp.int32)] * NI
        + [pltpu.VMEM((CH, D), jnp.float32)] * NB
        + [pltpu.VMEM_SHARED((NPAD, D), jnp.float32),
           pltpu.SemaphoreType.DMA,
           pltpu.SemaphoreType.DMA,
           pltpu.SemaphoreType.DMA]
    ),
)


def _sc_deg_body(idx_hbm, ones_hbm, zeros_hbm, out_hbm, i0, i1, ones_v,
                 acc_sh, isem):
    c = lax.axis_index("c")
    s = lax.axis_index("s")
    wid = s * NC + c
    idx = (i0, i1)
    pltpu.sync_copy(zeros_hbm, acc_sh.at[pl.ds(s * RPW, RPW)])
    pltpu.sync_copy(ones_hbm, ones_v)
    plsc.subcore_barrier()
    pltpu.async_copy(idx_hbm.at[wid, 0], i0, isem)
    pltpu.async_copy(idx_hbm.at[wid, 1], i1, isem)

    def stage(t, j):
        pltpu.make_async_copy(idx_hbm.at[wid, 0], idx[j], isem).wait()
        pltpu.sync_copy(ones_v, acc_sh.at[idx[j].at[1]], add=True)
        cond = not isinstance(t, int)
        if cond or t + 2 < T:
            pltpu.async_copy(idx_hbm.at[wid, t + 2], idx[j], isem)

    for t in range(2):
        stage(t, t)

    def middle(i, carry):
        for j in range(2):
            stage(i * 2 + j, j)
        return carry

    lax.fori_loop(1, T // 2 - 1, middle, 0)
    for t in range(T - 2, T):
        stage(t, t % 2)
    plsc.subcore_barrier()
    pltpu.sync_copy(acc_sh.at[pl.ds(s * RPW, RPW)],
                    out_hbm.at[pl.ds(c * NPAD + s * RPW, RPW)])


_sc_deg = pl.kernel(
    _sc_deg_body,
    out_type=jax.ShapeDtypeStruct((NC * NPAD,), jnp.float32),
    mesh=_MESH,
    scratch_types=[
        pltpu.VMEM((2, CH), jnp.int32),
        pltpu.VMEM((2, CH), jnp.int32),
        pltpu.VMEM((CH,), jnp.float32),
        pltpu.VMEM_SHARED((NPAD,), jnp.float32),
        pltpu.SemaphoreType.DMA,
    ],
)


def _bn_in(h, w, b):
    m = jnp.mean(h, axis=0)
    v = jnp.mean((h - m) ** 2, axis=0)
    return (h - m) / jnp.sqrt(v + EPS) * w + b


def _tc_a_body(x_ref, dega_ref, bnw_ref, bnb_ref, w_ref, y_ref, dis_ref):
    deg = dega_ref[0, :N] + dega_ref[1, :N] + 1.0
    dis = lax.rsqrt(deg)
    h = _bn_in(x_ref[...], bnw_ref[...], bnb_ref[...])
    xw = jnp.dot(h, w_ref[...], preferred_element_type=jnp.float32)
    y_ref[...] = xw * dis[:, None]
    dis_ref[...] = dis


def _tc_b_body(acc_ref, y_ref, dis_ref, bprev_ref, bnw_ref, bnb_ref, w_ref,
               yout_ref):
    dis = dis_ref[...]
    tot = acc_ref[0, :N, :] + acc_ref[1, :N, :] + y_ref[...]
    h = jnp.maximum(dis[:, None] * tot + bprev_ref[...], 0.0)
    hb = _bn_in(h, bnw_ref[...], bnb_ref[...])
    xw = jnp.dot(hb, w_ref[...], preferred_element_type=jnp.float32)
    yout_ref[...] = xw * dis[:, None]


def _tc_c_body(acc_ref, y_ref, dis_ref, bprev_ref, batch_ref, g1_ref, g1b_ref,
               g2_ref, g2b_ref, f1_ref, f1b_ref, bnw_ref, bnb_ref, wc_ref,
               bc_ref, out_ref):
    dis = dis_ref[...]
    tot = acc_ref[0, :N, :] + acc_ref[1, :N, :] + y_ref[...]
    h = jnp.maximum(dis[:, None] * tot + bprev_ref[...], 0.0)
    hg1 = jnp.maximum(
        jnp.dot(h, g1_ref[...], preferred_element_type=jnp.float32)
        + g1b_ref[...], 0.0)
    glin = jnp.dot(hg1, g2_ref[...], preferred_element_type=jnp.float32) \
        + g2b_ref[...]
    gate = jax.nn.sigmoid(glin)
    hg = h * gate
    onehot = (batch_ref[...][None, :]
              == lax.broadcasted_iota(jnp.int32, (G, N), 0)
              ).astype(jnp.float32)
    pooled = jnp.dot(onehot, hg, preferred_element_type=jnp.float32)
    pb = _bn_in(pooled, bnw_ref[...], bnb_ref[...])
    z = jnp.maximum(
        jnp.dot(pb, f1_ref[...], preferred_element_type=jnp.float32)
        + f1b_ref[...], 0.0)
    logits = jnp.dot(z, wc_ref[...], preferred_element_type=jnp.float32) \
        + bc_ref[...]
    lmax = jnp.max(logits, axis=-1, keepdims=True)
    e = logits - lmax
    out_ref[...] = e - jnp.log(jnp.sum(jnp.exp(e), axis=-1, keepdims=True))


def kernel(x, edge_index, batch, bnf_w, bnf_b, Wf, bf, W0, b0, bn0_w, bn0_b,
           W1, b1, bn1_w, bn1_b, W2, b2, bn2_w, bn2_b, G1, G1b, G2, G2b, F1,
           F1b, bnfc_w, bnfc_b, Wc, bc):
    src = edge_index[0].astype(jnp.int32)
    dst = edge_index[1].astype(jnp.int32)
    pad = EPAD - E
    pad_ar = jnp.arange(pad, dtype=jnp.int32)
    src_p = jnp.concatenate([src, pad_ar % 128]).reshape(NW, T, CH)
    dst_p = jnp.concatenate([dst, N + pad_ar % NTRASH]).reshape(NW, T, CH)
    idx_p = jnp.stack([src_p, dst_p], axis=2)

    zeros_d = jnp.zeros((NS, RPW, D), jnp.float32)

    dega = _sc_deg(idx_p, jnp.ones((CH,), jnp.float32),
                   jnp.zeros((RPW,), jnp.float32)).reshape(NC, NPAD)

    y0, dis = pl.pallas_call(
        _tc_a_body,
        out_shape=[jax.ShapeDtypeStruct((N, D), jnp.float32),
                   jax.ShapeDtypeStruct((N,), jnp.float32)],
    )(x, dega, bnf_w, bnf_b, Wf)

    tc_b = pl.pallas_call(
        _tc_b_body,
        out_shape=jax.ShapeDtypeStruct((N, D), jnp.float32),
    )

    y = y0
    for b_prev, bw, bb, W in ((bf, bn0_w, bn0_b, W0),
                              (b0, bn1_w, bn1_b, W1),
                              (b1, bn2_w, bn2_b, W2)):
        acc = _sc_scatter(y, idx_p, zeros_d)
        y = tc_b(acc, y, dis, b_prev, bw, bb, W)

    acc = _sc_scatter(y, idx_p, zeros_d)
    out = pl.pallas_call(
        _tc_c_body,
        out_shape=jax.ShapeDtypeStruct((G, C), jnp.float32),
    )(acc, y, dis, b2, batch, G1, G1b, G2, G2b, F1, F1b, bnfc_w, bnfc_b,
      Wc, bc)
    return out
